# baseline (speedup 1.0000x reference)
import functools

import numpy as np
import jax
import jax.numpy as jnp
from jax import lax
from jax.experimental import pallas as pl
from jax.experimental.pallas import tpu as pltpu
from jax.experimental.pallas import tpu_sc as plsc

_H = 2048
_W = 2048
_CH = 4
_NWORK = 32
_RPW = _H // _NWORK
_RPB = 8
_NB = _RPW // _RPB
_SEG = 128
_NCHUNKB = 96
_STRIPE = _NCHUNKB * 16

_NORM = np.sqrt(np.float32(4.0 * 4.0 + 1024.0 * 1024.0 + 1024.0 * 1024.0),
                dtype=np.float32)
_S = np.float32(np.float32(_NORM) * np.float32(0.5) / np.float32(2048.0))


def _prep_kernel(cos_ref, sin_ref, nv_ref):
    i = lax.broadcasted_iota(jnp.int32, (32, 128), 0)
    j = lax.broadcasted_iota(jnp.int32, (32, 128), 1)
    t = (i * 64 + (j & 63)).astype(jnp.float32)
    ang = t * 2.0 * np.float32(np.pi) / 2048.0
    c = jnp.cos(ang)
    s = jnp.sin(ang)
    cos_ref[...] = c
    sin_ref[...] = s
    asc = jnp.abs(c) * _S
    ass = jnp.abs(s) * _S
    bx = jnp.where(asc > 0.0, 512.0 / jnp.maximum(asc, 1e-30), 3000.0)
    by = jnp.where(ass > 0.0, 2.0 / jnp.maximum(ass, 1e-30), 3000.0)
    bound = jnp.minimum(jnp.minimum(bx, by), 3000.0)
    nv_ref[...] = jnp.clip((bound * (1.0 / 16.0)).astype(jnp.int32) + 3,
                           1, _NCHUNKB)


_prep = pl.pallas_call(
    _prep_kernel,
    out_shape=(jax.ShapeDtypeStruct((32, 128), jnp.float32),
               jax.ShapeDtypeStruct((32, 128), jnp.float32),
               jax.ShapeDtypeStruct((32, 128), jnp.int32)),
)


def _zero_kernel(o_ref):
    o_ref[...] = jnp.zeros_like(o_ref)


_zeros = pl.pallas_call(
    _zero_kernel,
    out_shape=jax.ShapeDtypeStruct((_CH, _H, _W), jnp.float32),
    grid=(16,),
    out_specs=pl.BlockSpec((_CH, _H // 16, _W), lambda i: (0, i, 0)),
)

_mesh = plsc.VectorSubcoreMesh(core_axis_name="c", subcore_axis_name="s")


@functools.partial(
    pl.kernel,
    mesh=_mesh,
    out_type=(),
    scratch_types=[
        pltpu.VMEM((_CH * 4 * 1024,), jnp.float32),
        pltpu.VMEM((1, 128), jnp.float32),
        pltpu.VMEM((1, 128), jnp.float32),
        pltpu.VMEM((1, 128), jnp.int32),
        pltpu.VMEM((2, _CH, _RPB, _STRIPE), jnp.float32),
        pltpu.SemaphoreType.DMA,
        pltpu.SemaphoreType.DMA,
    ],
    compiler_params=pltpu.CompilerParams(needs_layout_passes=False),
)
def _remap(tbl_hbm, cosb_hbm, sinb_hbm, nvb_hbm, out_ref,
           table_v, cos_v, sin_v, nv_v, buf_v, sem0, sem1):
    wid = lax.axis_index("s") * 2 + lax.axis_index("c")
    base = wid * _RPW
    pltpu.sync_copy(tbl_hbm, table_v)
    pltpu.sync_copy(cosb_hbm.at[pl.ds(wid, 1)], cos_v)
    pltpu.sync_copy(sinb_hbm.at[pl.ds(wid, 1)], sin_v)
    pltpu.sync_copy(nvb_hbm.at[pl.ds(wid, 1)], nv_v)
    iota16 = lax.iota(jnp.int32, 16)
    zeros16 = jnp.zeros((16,), jnp.float32)
    sems = (sem0, sem1)

    def drain(b, n):
        def dwait(j, c2):
            pltpu.make_async_copy(
                out_ref.at[:, pl.ds(0, _RPB), pl.ds(0, _SEG)],
                buf_v.at[b, :, :, pl.ds(0, _SEG)],
                sems[b]).wait()
            return c2
        lax.fori_loop(0, n, dwait, 0)

    cvecs = [cos_v[0, pl.ds(o, 16)] for o in range(0, _RPW, 16)]
    svecs = [sin_v[0, pl.ds(o, 16)] for o in range(0, _RPW, 16)]
    nvecs = [nv_v[0, pl.ds(o, 16)] for o in range(0, _RPW, 16)]

    def _at(vecs, lane):
        return vecs[lane // 16][lane % 16]

    prev_ms = [None, None]
    for bi in range(_NB):
        b = bi % 2
        rl0 = bi * _RPB
        row0 = base + rl0
        if bi >= 2:
            drain(b, prev_ms[b])

        nvs = [_at(nvecs, rl0 + i) for i in range(_RPB)]
        maxnv = functools.reduce(jnp.maximum, nvs)
        maxseg = lax.shift_right_logical(maxnv + 7, 3)
        kmax = maxseg * 8

        for i in range(_RPB):
            nv = nvs[i]
            cv = _at(cvecs, rl0 + i)
            sv = _at(svecs, rl0 + i)

            def chunk(k, c2, nv=nv, cv=cv, sv=sv, i=i, b=b):
                @pl.when(k < nv)
                def _():
                    rrv = (iota16 + k * 16).astype(jnp.float32)
                    rho = rrv * _S
                    x = 512.0 + rho * cv
                    y = 2.0 - rho * sv
                    m = (x >= 0.0) & (x < 1024.0) & (y >= 0.0) & (y < 4.0)
                    xi = jnp.clip(x.astype(jnp.int32), 0, 1023)
                    yi = jnp.clip(y.astype(jnp.int32), 0, 3)
                    idx = yi * 1024 + xi
                    for c in range(_CH):
                        val = plsc.load_gather(table_v, [idx + (c * 4096)])
                        buf_v[b, c, i, pl.ds(k * 16, 16)] = \
                            jnp.where(m, val, 0.0)

                @pl.when(k >= nv)
                def _():
                    for c in range(_CH):
                        buf_v[b, c, i, pl.ds(k * 16, 16)] = zeros16

                return c2

            lax.fori_loop(0, kmax, chunk, 0)

        def seg(j, c2, b=b, row0=row0):
            pltpu.async_copy(
                buf_v.at[b, :, :, pl.ds(j * _SEG, _SEG)],
                out_ref.at[:, pl.ds(row0, _RPB), pl.ds(j * _SEG, _SEG)],
                sems[b])
            return c2

        lax.fori_loop(0, maxseg, seg, 0)
        prev_ms[b] = maxseg

    drain(0, prev_ms[0])
    drain(1, prev_ms[1])


def kernel(data):
    cos_b, sin_b, nv_b = _prep()
    tbl = data[:, :4, :].reshape(_CH * 4 * 1024)
    z = _zeros()
    zref = jax.new_ref(z)
    _remap(tbl, cos_b, sin_b, nv_b, zref)
    return zref[...]


# speedup vs baseline: 1.1672x; 1.1672x over previous
import functools

import numpy as np
import jax
import jax.numpy as jnp
from jax import lax
from jax.experimental import pallas as pl
from jax.experimental.pallas import tpu as pltpu
from jax.experimental.pallas import tpu_sc as plsc

_H = 2048
_W = 2048
_CH = 4
_NWORK = 32
_RPW = _H // _NWORK
_RPB = 4
_NB = _RPW // _RPB
_SEG = 128
_NCHUNKB = 96
_STRIPE = _NCHUNKB * 16

_NORM = np.sqrt(np.float32(4.0 * 4.0 + 1024.0 * 1024.0 + 1024.0 * 1024.0),
                dtype=np.float32)
_S = np.float32(np.float32(_NORM) * np.float32(0.5) / np.float32(2048.0))


def _fill_kernel(z_ref, cos_ref, sin_ref, nv_ref):
    z_ref[...] = jnp.zeros_like(z_ref)

    @pl.when(pl.program_id(0) == 0)
    def _():
        t = lax.broadcasted_iota(jnp.int32, (_H, 16), 0).astype(jnp.float32)
        ang = t * 2.0 * np.float32(np.pi) / 2048.0
        c = jnp.cos(ang)
        s = jnp.sin(ang)
        cos_ref[...] = c
        sin_ref[...] = s
        asc = jnp.abs(c) * _S
        ass = jnp.abs(s) * _S
        bx = jnp.where(asc > 0.0, 512.0 / jnp.maximum(asc, 1e-30), 3000.0)
        by = jnp.where(ass > 0.0, 2.0 / jnp.maximum(ass, 1e-30), 3000.0)
        bound = jnp.minimum(jnp.minimum(bx, by), 3000.0)
        nv_ref[...] = jnp.clip((bound * (1.0 / 16.0)).astype(jnp.int32) + 3,
                               1, _NCHUNKB)


_fill = pl.pallas_call(
    _fill_kernel,
    out_shape=(jax.ShapeDtypeStruct((_CH, _H, _W), jnp.float32),
               jax.ShapeDtypeStruct((_H, 16), jnp.float32),
               jax.ShapeDtypeStruct((_H, 16), jnp.float32),
               jax.ShapeDtypeStruct((_H, 16), jnp.int32)),
    grid=(15,),
    out_specs=(
        pl.BlockSpec((_CH, _H, _SEG), lambda j: (0, 0, j + 1)),
        pl.BlockSpec((_H, 16), lambda j: (0, 0)),
        pl.BlockSpec((_H, 16), lambda j: (0, 0)),
        pl.BlockSpec((_H, 16), lambda j: (0, 0)),
    ),
)

_mesh = plsc.VectorSubcoreMesh(core_axis_name="c", subcore_axis_name="s")


@functools.partial(
    pl.kernel,
    mesh=_mesh,
    out_type=(),
    scratch_types=[
        pltpu.VMEM((_CH * 4 * 1024,), jnp.float32),
        pltpu.VMEM((_RPW, 16), jnp.float32),
        pltpu.VMEM((_RPW, 16), jnp.float32),
        pltpu.VMEM((_RPW, 16), jnp.int32),
        pltpu.VMEM((2, _CH, _RPB, _STRIPE), jnp.float32),
        pltpu.SemaphoreType.DMA,
        pltpu.SemaphoreType.DMA,
    ],
    compiler_params=pltpu.CompilerParams(needs_layout_passes=False),
)
def _remap(tbl_hbm, cosb_hbm, sinb_hbm, nvb_hbm, out_ref,
           table_v, cos_v, sin_v, nv_v, buf_v, sem0, sem1):
    wid = lax.axis_index("s") * 2 + lax.axis_index("c")
    base = wid * _RPW
    pltpu.sync_copy(tbl_hbm, table_v)
    pltpu.sync_copy(cosb_hbm.at[pl.ds(base, _RPW)], cos_v)
    pltpu.sync_copy(sinb_hbm.at[pl.ds(base, _RPW)], sin_v)
    pltpu.sync_copy(nvb_hbm.at[pl.ds(base, _RPW)], nv_v)
    iota16 = lax.iota(jnp.int32, 16)
    sems = (sem0, sem1)

    def drain(b, n):
        def dwait(j, c2):
            pltpu.make_async_copy(
                out_ref.at[:, pl.ds(0, _RPB), pl.ds(0, _SEG)],
                buf_v.at[b, :, :, pl.ds(0, _SEG)],
                sems[b]).wait()
            return c2
        lax.fori_loop(0, n, dwait, 0)

    def group(g, carry):
        prev = (carry[0], carry[1])
        new = [None, None]
        for b in range(2):
            rl0 = (2 * g + b) * _RPB
            row0 = base + rl0

            @pl.when(g > 0)
            def _(b=b):
                drain(b, prev[b])

            nvv = nv_v[rl0]
            cvs, svs = [], []
            for i in range(_RPB):
                if i:
                    nvv = jnp.maximum(nvv, nv_v[rl0 + i])
                cvs.append(cos_v[rl0 + i])
                svs.append(sin_v[rl0 + i])
            maxseg = lax.shift_right_logical(jnp.max(nvv) + 7, 3)

            def chunk(k, c2, b=b, cvs=cvs, svs=svs):
                rrv = (iota16 + k * 16).astype(jnp.float32)
                rho = rrv * _S
                for i in range(_RPB):
                    x = 512.0 + rho * cvs[i]
                    y = 2.0 - rho * svs[i]
                    m = (x >= 0.0) & (x < 1024.0) & (y >= 0.0) & (y < 4.0)
                    xi = jnp.clip(x.astype(jnp.int32), 0, 1023)
                    yi = jnp.clip(y.astype(jnp.int32), 0, 3)
                    idx = yi * 1024 + xi
                    for c in range(_CH):
                        val = plsc.load_gather(table_v, [idx + (c * 4096)])
                        buf_v[b, c, i, pl.ds(k * 16, 16)] = \
                            jnp.where(m, val, 0.0)
                return c2

            lax.fori_loop(0, maxseg * 8, chunk, 0)

            def seg(j, c2, b=b, row0=row0):
                pltpu.async_copy(
                    buf_v.at[b, :, :, pl.ds(j * _SEG, _SEG)],
                    out_ref.at[:, pl.ds(row0, _RPB), pl.ds(j * _SEG, _SEG)],
                    sems[b])
                return c2

            lax.fori_loop(0, maxseg, seg, 0)
            new[b] = maxseg
        return (new[0], new[1])

    fin = lax.fori_loop(0, _NB // 2, group, (jnp.int32(0), jnp.int32(0)))
    drain(0, fin[0])
    drain(1, fin[1])


def kernel(data):
    tbl = data[:, :4, :].reshape(_CH * 4 * 1024)
    z, cos_b, sin_b, nv_b = _fill()
    zref = jax.new_ref(z)
    _remap(tbl, cos_b, sin_b, nv_b, zref)
    return zref[...]


# speedup vs baseline: 1.1767x; 1.0082x over previous
import functools

import numpy as np
import jax
import jax.numpy as jnp
from jax import lax
from jax.experimental import pallas as pl
from jax.experimental.pallas import tpu as pltpu
from jax.experimental.pallas import tpu_sc as plsc

_H = 2048
_W = 2048
_CH = 4
_NWORK = 32
_RPW = _H // _NWORK
_RPB = 4
_NB = _RPW // _RPB
_SEG = 128
_NCHUNKB = 96
_STRIPE = _NCHUNKB * 16

_PERM = np.zeros((_H,), np.int32)
for _w in range(_NWORK):
    for _j in range(_NB):
        _s = _w * _RPW + _j * _RPB
        _t0 = 64 * _j + 2 * _w
        _PERM[_s:_s + 4] = (_t0, _t0 + 1, _t0 + 1024, _t0 + 1025)
assert np.array_equal(np.sort(_PERM), np.arange(_H))

_NORM = np.sqrt(np.float32(4.0 * 4.0 + 1024.0 * 1024.0 + 1024.0 * 1024.0),
                dtype=np.float32)
_S = np.float32(np.float32(_NORM) * np.float32(0.5) / np.float32(2048.0))


def _fill_kernel(perm_ref, z_ref, cos_ref, sin_ref, nv_ref):
    z_ref[...] = jnp.zeros_like(z_ref)

    @pl.when(pl.program_id(0) == 0)
    def _():
        t = perm_ref[...].astype(jnp.float32)
        ang = t * 2.0 * np.float32(np.pi) / 2048.0
        c = jnp.cos(ang)
        s = jnp.sin(ang)
        cos_ref[...] = c
        sin_ref[...] = s
        asc = jnp.abs(c) * _S
        ass = jnp.abs(s) * _S
        bx = jnp.where(asc > 0.0, 512.0 / jnp.maximum(asc, 1e-30), 3000.0)
        by = jnp.where(ass > 0.0, 2.0 / jnp.maximum(ass, 1e-30), 3000.0)
        bound = jnp.minimum(jnp.minimum(bx, by), 3000.0)
        nv_ref[...] = jnp.clip((bound * (1.0 / 16.0)).astype(jnp.int32) + 3,
                               1, _NCHUNKB)


_fill = pl.pallas_call(
    _fill_kernel,
    out_shape=(jax.ShapeDtypeStruct((_CH, _H, _W), jnp.float32),
               jax.ShapeDtypeStruct((_H, 16), jnp.float32),
               jax.ShapeDtypeStruct((_H, 16), jnp.float32),
               jax.ShapeDtypeStruct((_H, 16), jnp.int32)),
    grid=(15,),
    in_specs=(pl.BlockSpec((_H, 16), lambda j: (0, 0)),),
    out_specs=(
        pl.BlockSpec((_CH, _H, _SEG), lambda j: (0, 0, j + 1)),
        pl.BlockSpec((_H, 16), lambda j: (0, 0)),
        pl.BlockSpec((_H, 16), lambda j: (0, 0)),
        pl.BlockSpec((_H, 16), lambda j: (0, 0)),
    ),
)

_mesh = plsc.VectorSubcoreMesh(core_axis_name="c", subcore_axis_name="s")


@functools.partial(
    pl.kernel,
    mesh=_mesh,
    out_type=(),
    scratch_types=[
        pltpu.VMEM((_CH * 4 * 1024,), jnp.float32),
        pltpu.VMEM((_RPW, 16), jnp.float32),
        pltpu.VMEM((_RPW, 16), jnp.float32),
        pltpu.VMEM((_RPW, 16), jnp.int32),
        pltpu.VMEM((2, _CH, _RPB, _STRIPE), jnp.float32),
        pltpu.SemaphoreType.DMA,
        pltpu.SemaphoreType.DMA,
    ],
    compiler_params=pltpu.CompilerParams(needs_layout_passes=False),
)
def _remap(tbl_hbm, cosb_hbm, sinb_hbm, nvb_hbm, out_ref,
           table_v, cos_v, sin_v, nv_v, buf_v, sem0, sem1):
    wid = lax.axis_index("s") * 2 + lax.axis_index("c")
    base = wid * _RPW
    pltpu.sync_copy(tbl_hbm, table_v)
    pltpu.sync_copy(cosb_hbm.at[pl.ds(base, _RPW)], cos_v)
    pltpu.sync_copy(sinb_hbm.at[pl.ds(base, _RPW)], sin_v)
    pltpu.sync_copy(nvb_hbm.at[pl.ds(base, _RPW)], nv_v)
    iota16 = lax.iota(jnp.int32, 16)
    sems = (sem0, sem1)

    def drain(b, n):
        def dwait(j, c2):
            pltpu.make_async_copy(
                out_ref.at[:, pl.ds(0, _RPB), pl.ds(0, _SEG)],
                buf_v.at[b, :, :, pl.ds(0, _SEG)],
                sems[b]).wait()
            return c2
        lax.fori_loop(0, n, dwait, 0)

    def group(g, carry):
        prev = (carry[0], carry[1])
        new = [None, None]
        for b in range(2):
            bi = 2 * g + b
            rl0 = bi * _RPB
            t0 = 64 * bi + 2 * wid

            @pl.when(g > 0)
            def _(b=b):
                drain(b, prev[b])

            nvv = nv_v[rl0]
            cvs, svs = [], []
            for i in range(_RPB):
                if i:
                    nvv = jnp.maximum(nvv, nv_v[rl0 + i])
                cvs.append(cos_v[rl0 + i])
                svs.append(sin_v[rl0 + i])
            maxseg = lax.shift_right_logical(jnp.max(nvv) + 7, 3)

            def chunk(k, c2, b=b, cvs=cvs, svs=svs):
                rrv = (iota16 + k * 16).astype(jnp.float32)
                rho = rrv * _S
                for i in range(_RPB):
                    x = 512.0 + rho * cvs[i]
                    y = 2.0 - rho * svs[i]
                    m = (x >= 0.0) & (x < 1024.0) & (y >= 0.0) & (y < 4.0)
                    xi = jnp.clip(x.astype(jnp.int32), 0, 1023)
                    yi = jnp.clip(y.astype(jnp.int32), 0, 3)
                    idx = yi * 1024 + xi
                    for c in range(_CH):
                        val = plsc.load_gather(table_v, [idx + (c * 4096)])
                        buf_v[b, c, i, pl.ds(k * 16, 16)] = \
                            jnp.where(m, val, 0.0)
                return c2

            lax.fori_loop(0, maxseg * 8, chunk, 0)

            def seg(j, c2, b=b, t0=t0):
                pltpu.async_copy(
                    buf_v.at[b, :, pl.ds(0, 2), pl.ds(j * _SEG, _SEG)],
                    out_ref.at[:, pl.ds(t0, 2), pl.ds(j * _SEG, _SEG)],
                    sems[b])
                pltpu.async_copy(
                    buf_v.at[b, :, pl.ds(2, 2), pl.ds(j * _SEG, _SEG)],
                    out_ref.at[:, pl.ds(t0 + 1024, 2), pl.ds(j * _SEG, _SEG)],
                    sems[b])
                return c2

            lax.fori_loop(0, maxseg, seg, 0)
            new[b] = maxseg
        return (new[0], new[1])

    fin = lax.fori_loop(0, _NB // 2, group, (jnp.int32(0), jnp.int32(0)))
    drain(0, fin[0])
    drain(1, fin[1])


def kernel(data):
    tbl = data[:, :4, :].reshape(_CH * 4 * 1024)
    perm_b = jnp.broadcast_to(jnp.asarray(_PERM)[:, None], (_H, 16))
    z, cos_b, sin_b, nv_b = _fill(perm_b)
    zref = jax.new_ref(z)
    _remap(tbl, cos_b, sin_b, nv_b, zref)
    return zref[...]


# speedup vs baseline: 1.2457x; 1.0586x over previous
import functools

import numpy as np
import jax
import jax.numpy as jnp
from jax import lax
from jax.experimental import pallas as pl
from jax.experimental.pallas import tpu as pltpu
from jax.experimental.pallas import tpu_sc as plsc

_H = 2048
_W = 2048
_CH = 4
_NWORK = 32
_RPW = _H // _NWORK
_RPB = 4
_NB = _RPW // _RPB
_SEG = 128
_NCHUNKB = 96
_STRIPE = _NCHUNKB * 16

_PERM = np.zeros((_H,), np.int32)
for _w in range(_NWORK):
    for _j in range(_NB):
        _s = _w * _RPW + _j * _RPB
        _t0 = 64 * _j + 2 * _w
        _PERM[_s:_s + 4] = (_t0, _t0 + 1, _t0 + 1024, _t0 + 1025)
assert np.array_equal(np.sort(_PERM), np.arange(_H))

_NORM = np.sqrt(np.float32(4.0 * 4.0 + 1024.0 * 1024.0 + 1024.0 * 1024.0),
                dtype=np.float32)
_S = np.float32(np.float32(_NORM) * np.float32(0.5) / np.float32(2048.0))


def _fill_kernel(perm_ref, z_ref, cos_ref, sin_ref, nv_ref):
    z_ref[...] = jnp.zeros_like(z_ref)

    @pl.when(pl.program_id(0) == 0)
    def _():
        t = perm_ref[...].astype(jnp.float32)
        ang = t * 2.0 * np.float32(np.pi) / 2048.0
        c = jnp.cos(ang)
        s = jnp.sin(ang)
        cos_ref[...] = c
        sin_ref[...] = s
        asc = jnp.abs(c) * _S
        ass = jnp.abs(s) * _S
        bx = jnp.where(asc > 0.0, 512.0 / jnp.maximum(asc, 1e-30), 3000.0)
        by = jnp.where(ass > 0.0, 2.0 / jnp.maximum(ass, 1e-30), 3000.0)
        bound = jnp.minimum(jnp.minimum(bx, by), 3000.0)
        nv_ref[...] = jnp.clip((bound * (1.0 / 16.0)).astype(jnp.int32) + 3,
                               1, _NCHUNKB)


_fill = pl.pallas_call(
    _fill_kernel,
    out_shape=(jax.ShapeDtypeStruct((_CH, _H, _W), jnp.float32),
               jax.ShapeDtypeStruct((_H, 16), jnp.float32),
               jax.ShapeDtypeStruct((_H, 16), jnp.float32),
               jax.ShapeDtypeStruct((_H, 16), jnp.int32)),
    grid=(4,),
    in_specs=(pl.BlockSpec((_H, 16), lambda j: (0, 0)),),
    out_specs=(
        pl.BlockSpec((_CH, _H, _SEG), lambda j: (0, 0, j + 12)),
        pl.BlockSpec((_H, 16), lambda j: (0, 0)),
        pl.BlockSpec((_H, 16), lambda j: (0, 0)),
        pl.BlockSpec((_H, 16), lambda j: (0, 0)),
    ),
)

_mesh = plsc.VectorSubcoreMesh(core_axis_name="c", subcore_axis_name="s")


@functools.partial(
    pl.kernel,
    mesh=_mesh,
    out_type=(),
    scratch_types=[
        pltpu.VMEM((_CH * 4 * 1024,), jnp.float32),
        pltpu.VMEM((_RPW, 16), jnp.float32),
        pltpu.VMEM((_RPW, 16), jnp.float32),
        pltpu.VMEM((_RPW, 16), jnp.int32),
        pltpu.VMEM((2, _CH, _RPB, _STRIPE), jnp.float32),
        pltpu.SemaphoreType.DMA,
        pltpu.SemaphoreType.DMA,
    ],
    compiler_params=pltpu.CompilerParams(needs_layout_passes=False),
)
def _remap(tbl_hbm, cosb_hbm, sinb_hbm, nvb_hbm, out_ref,
           table_v, cos_v, sin_v, nv_v, buf_v, sem0, sem1):
    wid = lax.axis_index("s") * 2 + lax.axis_index("c")
    base = wid * _RPW
    pltpu.sync_copy(tbl_hbm, table_v)
    pltpu.sync_copy(cosb_hbm.at[pl.ds(base, _RPW)], cos_v)
    pltpu.sync_copy(sinb_hbm.at[pl.ds(base, _RPW)], sin_v)
    pltpu.sync_copy(nvb_hbm.at[pl.ds(base, _RPW)], nv_v)
    iota16 = lax.iota(jnp.int32, 16)
    zeros16 = jnp.zeros((16,), jnp.float32)
    sems = (sem0, sem1)

    def drain(b):
        pltpu.make_async_copy(
            out_ref.at[:, pl.ds(0, _RPB), pl.ds(0, _STRIPE)],
            buf_v.at[b],
            sems[b]).wait()

    def group(g, carry):
        prev = (carry[0], carry[1])
        new = [None, None]
        for b in range(2):
            bi = 2 * g + b
            rl0 = bi * _RPB
            t0 = 64 * bi + 2 * wid

            @pl.when(g > 0)
            def _(b=b):
                drain(b)

            nvv = nv_v[rl0]
            cvs, svs = [], []
            for i in range(_RPB):
                if i:
                    nvv = jnp.maximum(nvv, nv_v[rl0 + i])
                cvs.append(cos_v[rl0 + i])
                svs.append(sin_v[rl0 + i])
            maxseg = lax.shift_right_logical(jnp.max(nvv) + 7, 3)
            kmax = maxseg * 8

            def zchunk(k, c2, b=b):
                for i in range(_RPB):
                    for c in range(_CH):
                        buf_v[b, c, i, pl.ds(k * 16, 16)] = zeros16
                return c2

            lax.fori_loop(kmax, prev[b] * 8, zchunk, 0)

            def chunk(k, c2, b=b, cvs=cvs, svs=svs):
                rrv = (iota16 + k * 16).astype(jnp.float32)
                rho = rrv * _S
                for i in range(_RPB):
                    x = 512.0 + rho * cvs[i]
                    y = 2.0 - rho * svs[i]
                    m = (x >= 0.0) & (x < 1024.0) & (y >= 0.0) & (y < 4.0)
                    xi = jnp.clip(x.astype(jnp.int32), 0, 1023)
                    yi = jnp.clip(y.astype(jnp.int32), 0, 3)
                    idx = yi * 1024 + xi
                    for c in range(_CH):
                        val = plsc.load_gather(table_v, [idx + (c * 4096)])
                        buf_v[b, c, i, pl.ds(k * 16, 16)] = \
                            jnp.where(m, val, 0.0)
                return c2

            lax.fori_loop(0, kmax, chunk, 0)

            pltpu.async_copy(
                buf_v.at[b, :, pl.ds(0, 2), :],
                out_ref.at[:, pl.ds(t0, 2), pl.ds(0, _STRIPE)],
                sems[b])
            pltpu.async_copy(
                buf_v.at[b, :, pl.ds(2, 2), :],
                out_ref.at[:, pl.ds(t0 + 1024, 2), pl.ds(0, _STRIPE)],
                sems[b])
            new[b] = maxseg
        return (new[0], new[1])

    fin = lax.fori_loop(0, _NB // 2, group,
                        (jnp.int32(_NCHUNKB // 8), jnp.int32(_NCHUNKB // 8)))
    drain(0)
    drain(1)


def kernel(data):
    tbl = data[:, :4, :].reshape(_CH * 4 * 1024)
    perm_b = jnp.broadcast_to(jnp.asarray(_PERM)[:, None], (_H, 16))
    z, cos_b, sin_b, nv_b = _fill(perm_b)
    zref = jax.new_ref(z)
    _remap(tbl, cos_b, sin_b, nv_b, zref)
    return zref[...]


# speedup vs baseline: 1.4170x; 1.1376x over previous
import functools

import numpy as np
import jax
import jax.numpy as jnp
from jax import lax
from jax.experimental import pallas as pl
from jax.experimental.pallas import tpu as pltpu
from jax.experimental.pallas import tpu_sc as plsc

_H = 2048
_W = 2048
_CH = 4
_NWORK = 32
_RPW = _H // _NWORK
_NCHUNK = _W // 16

_NORM = np.sqrt(np.float32(4.0 * 4.0 + 1024.0 * 1024.0 + 1024.0 * 1024.0),
                dtype=np.float32)
_S = np.float32(np.float32(_NORM) * np.float32(0.5) / np.float32(2048.0))


def _trig_kernel(cos_ref, sin_ref, nv_ref):
    i = lax.broadcasted_iota(jnp.int32, (16, 128), 0)
    j = lax.broadcasted_iota(jnp.int32, (16, 128), 1)
    t = (i * 128 + j).astype(jnp.float32)
    ang = t * 2.0 * np.float32(np.pi) / 2048.0
    c = jnp.cos(ang)
    s = jnp.sin(ang)
    cos_ref[...] = c
    sin_ref[...] = s
    asc = jnp.abs(c) * _S
    ass = jnp.abs(s) * _S
    bx = jnp.where(asc > 0.0, 512.0 / jnp.maximum(asc, 1e-30), 3000.0)
    by = jnp.where(ass > 0.0, 2.0 / jnp.maximum(ass, 1e-30), 3000.0)
    bound = jnp.minimum(jnp.minimum(bx, by), 3000.0)
    nv_ref[...] = jnp.clip((bound * (1.0 / 16.0)).astype(jnp.int32) + 3,
                           1, _NCHUNK)


_trig = pl.pallas_call(
    _trig_kernel,
    out_shape=(jax.ShapeDtypeStruct((16, 128), jnp.float32),
               jax.ShapeDtypeStruct((16, 128), jnp.float32),
               jax.ShapeDtypeStruct((16, 128), jnp.int32)),
)

_mesh = plsc.VectorSubcoreMesh(core_axis_name="c", subcore_axis_name="s")


@functools.partial(
    pl.kernel,
    mesh=_mesh,
    out_type=jax.ShapeDtypeStruct((_CH, _H, _W), jnp.float32),
    scratch_types=[
        pltpu.VMEM((_CH * 4 * 1024,), jnp.float32),
        pltpu.VMEM((_RPW, 16), jnp.float32),
        pltpu.VMEM((_RPW, 16), jnp.float32),
        pltpu.VMEM((_RPW, 16), jnp.int32),
        pltpu.VMEM((2, _CH, 1, _W), jnp.float32),
        pltpu.SemaphoreType.DMA,
        pltpu.SemaphoreType.DMA,
    ],
    compiler_params=pltpu.CompilerParams(needs_layout_passes=False),
)
def _remap(tbl_hbm, cosb_hbm, sinb_hbm, nvb_hbm, out_hbm,
           table_v, cos_v, sin_v, nv_v, buf_v, sem0, sem1):
    wid = lax.axis_index("s") * 2 + lax.axis_index("c")
    base = wid * _RPW
    pltpu.sync_copy(tbl_hbm, table_v)
    pltpu.sync_copy(cosb_hbm.at[pl.ds(base, _RPW)], cos_v)
    pltpu.sync_copy(sinb_hbm.at[pl.ds(base, _RPW)], sin_v)
    pltpu.sync_copy(nvb_hbm.at[pl.ds(base, _RPW)], nv_v)
    iota16 = lax.iota(jnp.int32, 16)
    zeros16 = jnp.zeros((16,), jnp.float32)
    sems = (sem0, sem1)

    def group(g, carry):
        carry = list(carry)
        for b in range(2):
            rl = 2 * g + b
            row = base + rl

            @pl.when(g > 0)
            def _wait():
                pltpu.make_async_copy(out_hbm.at[:, pl.ds(row, 1), :],
                                      buf_v.at[b], sems[b]).wait()

            def zchunk(k, c2):
                for c in range(_CH):
                    buf_v[b, c, 0, pl.ds(k * 16, 16)] = zeros16
                return c2

            lax.fori_loop(0, carry[b], zchunk, 0)

            cv = cos_v[rl]
            sv = sin_v[rl]
            nv = jnp.max(nv_v[rl])

            def chunk(k, c2):
                rrv = (iota16 + k * 16).astype(jnp.float32)
                rho = rrv * _S
                x = 512.0 + rho * cv
                y = 2.0 - rho * sv
                m = (x >= 0.0) & (x < 1024.0) & (y >= 0.0) & (y < 4.0)
                xi = jnp.clip(x.astype(jnp.int32), 0, 1023)
                yi = jnp.clip(y.astype(jnp.int32), 0, 3)
                idx = yi * 1024 + xi
                for c in range(_CH):
                    val = plsc.load_gather(table_v, [idx + (c * 4096)])
                    buf_v[b, c, 0, pl.ds(k * 16, 16)] = jnp.where(m, val, 0.0)
                return c2

            lax.fori_loop(0, nv, chunk, 0)
            pltpu.async_copy(buf_v.at[b], out_hbm.at[:, pl.ds(row, 1), :],
                             sems[b])
            carry[b] = nv
        return tuple(carry)

    lax.fori_loop(0, _RPW // 2, group,
                  (jnp.int32(_NCHUNK), jnp.int32(_NCHUNK)))
    for b in range(2):
        pltpu.make_async_copy(out_hbm.at[:, pl.ds(base, 1), :],
                              buf_v.at[b], sems[b]).wait()


def kernel(data):
    cos_t, sin_t, nv_t = _trig()
    cos_b = jnp.broadcast_to(cos_t.reshape(_H, 1), (_H, 16))
    sin_b = jnp.broadcast_to(sin_t.reshape(_H, 1), (_H, 16))
    nv_b = jnp.broadcast_to(nv_t.reshape(_H, 1), (_H, 16))
    tbl = data[:, :4, :].reshape(_CH * 4 * 1024)
    return _remap(tbl, cos_b, sin_b, nv_b)


# speedup vs baseline: 1.4857x; 1.0484x over previous
import functools

import numpy as np
import jax
import jax.numpy as jnp
from jax import lax
from jax.experimental import pallas as pl
from jax.experimental.pallas import tpu as pltpu
from jax.experimental.pallas import tpu_sc as plsc

_H = 2048
_W = 2048
_CH = 4
_NWORK = 32
_RPW = _H // _NWORK
_NCHUNK = _W // 16

_NORM = np.sqrt(np.float32(4.0 * 4.0 + 1024.0 * 1024.0 + 1024.0 * 1024.0),
                dtype=np.float32)
_S = np.float32(np.float32(_NORM) * np.float32(0.5) / np.float32(2048.0))


def _trig_kernel(cos_ref, sin_ref, nv_ref):
    i = lax.broadcasted_iota(jnp.int32, (16, 128), 0)
    j = lax.broadcasted_iota(jnp.int32, (16, 128), 1)
    t = (i * 128 + j).astype(jnp.float32)
    ang = t * 2.0 * np.float32(np.pi) / 2048.0
    c = jnp.cos(ang)
    s = jnp.sin(ang)
    cos_ref[...] = c
    sin_ref[...] = s
    asc = jnp.abs(c) * _S
    ass = jnp.abs(s) * _S
    bx = jnp.where(asc > 0.0, 512.0 / jnp.maximum(asc, 1e-30), 3000.0)
    by = jnp.where(ass > 0.0, 2.0 / jnp.maximum(ass, 1e-30), 3000.0)
    bound = jnp.minimum(jnp.minimum(bx, by), 3000.0)
    nv_ref[...] = jnp.clip((bound * (1.0 / 16.0)).astype(jnp.int32) + 3,
                           1, _NCHUNK)


_trig = pl.pallas_call(
    _trig_kernel,
    out_shape=(jax.ShapeDtypeStruct((16, 128), jnp.float32),
               jax.ShapeDtypeStruct((16, 128), jnp.float32),
               jax.ShapeDtypeStruct((16, 128), jnp.int32)),
)

_mesh = plsc.VectorSubcoreMesh(core_axis_name="c", subcore_axis_name="s")


@functools.partial(
    pl.kernel,
    mesh=_mesh,
    out_type=jax.ShapeDtypeStruct((_CH, _H, _W), jnp.float32),
    scratch_types=[
        pltpu.VMEM((_CH * 4 * 1024,), jnp.float32),
        pltpu.VMEM((_RPW, 16), jnp.float32),
        pltpu.VMEM((_RPW, 16), jnp.float32),
        pltpu.VMEM((_RPW, 16), jnp.float32),
        pltpu.VMEM((2, _CH, 1, _W), jnp.float32),
        pltpu.SemaphoreType.DMA,
        pltpu.SemaphoreType.DMA,
    ],
    compiler_params=pltpu.CompilerParams(needs_layout_passes=False),
)
def _remap(tbl_hbm, tabs_hbm, out_hbm,
           table_v, cos_v, sin_v, nv_v, buf_v, sem0, sem1):
    wid = lax.axis_index("s") * 2 + lax.axis_index("c")
    base = wid * _RPW
    pltpu.sync_copy(tbl_hbm, table_v)
    pltpu.sync_copy(tabs_hbm.at[pl.ds(base, _RPW)], cos_v)
    pltpu.sync_copy(tabs_hbm.at[pl.ds(_H + base, _RPW)], sin_v)
    pltpu.sync_copy(tabs_hbm.at[pl.ds(2 * _H + base, _RPW)], nv_v)
    iota16 = lax.iota(jnp.int32, 16)
    zeros16 = jnp.zeros((16,), jnp.float32)
    sems = (sem0, sem1)

    def group(g, carry):
        carry = list(carry)
        for b in range(2):
            rl = 2 * g + b
            row = base + rl

            @pl.when(g > 0)
            def _wait():
                pltpu.make_async_copy(out_hbm.at[:, pl.ds(row, 1), :],
                                      buf_v.at[b], sems[b]).wait()

            def zchunk(k, c2):
                for c in range(_CH):
                    buf_v[b, c, 0, pl.ds(k * 16, 16)] = zeros16
                return c2

            lax.fori_loop(0, carry[b], zchunk, 0)

            cv = cos_v[rl]
            sv = sin_v[rl]
            nv = jnp.max(lax.bitcast_convert_type(nv_v[rl], jnp.int32))

            def chunk(k, c2):
                rrv = (iota16 + k * 16).astype(jnp.float32)
                rho = rrv * _S
                x = 512.0 + rho * cv
                y = 2.0 - rho * sv
                m = (x >= 0.0) & (x < 1024.0) & (y >= 0.0) & (y < 4.0)
                xi = jnp.clip(x.astype(jnp.int32), 0, 1023)
                yi = jnp.clip(y.astype(jnp.int32), 0, 3)
                idx = yi * 1024 + xi
                for c in range(_CH):
                    val = plsc.load_gather(table_v, [idx + (c * 4096)])
                    buf_v[b, c, 0, pl.ds(k * 16, 16)] = jnp.where(m, val, 0.0)
                return c2

            lax.fori_loop(0, nv, chunk, 0)
            pltpu.async_copy(buf_v.at[b], out_hbm.at[:, pl.ds(row, 1), :],
                             sems[b])
            carry[b] = nv
        return tuple(carry)

    lax.fori_loop(0, _RPW // 2, group,
                  (jnp.int32(_NCHUNK), jnp.int32(_NCHUNK)))
    for b in range(2):
        pltpu.make_async_copy(out_hbm.at[:, pl.ds(base, 1), :],
                              buf_v.at[b], sems[b]).wait()


def kernel(data):
    cos_t, sin_t, nv_t = _trig()
    nv_f = lax.bitcast_convert_type(nv_t, jnp.float32)
    tabs = jnp.concatenate(
        [cos_t.reshape(_H), sin_t.reshape(_H), nv_f.reshape(_H)])
    tabs_b = jnp.broadcast_to(tabs[:, None], (3 * _H, 16))
    tbl = data[:, :4, :].reshape(_CH * 4 * 1024)
    return _remap(tbl, tabs_b)


# speedup vs baseline: 1.4958x; 1.0068x over previous
import functools

import numpy as np
import jax
import jax.numpy as jnp
from jax import lax
from jax.experimental import pallas as pl
from jax.experimental.pallas import tpu as pltpu
from jax.experimental.pallas import tpu_sc as plsc

_H = 2048
_W = 2048
_CH = 4
_NWORK = 32
_RPW = _H // _NWORK
_NCHUNK = _W // 16

_NORM = np.sqrt(np.float32(4.0 * 4.0 + 1024.0 * 1024.0 + 1024.0 * 1024.0),
                dtype=np.float32)
_S = np.float32(np.float32(_NORM) * np.float32(0.5) / np.float32(2048.0))


def _trig_kernel(cos_ref, sin_ref, nv_ref):
    i = lax.broadcasted_iota(jnp.int32, (16, 128), 0)
    j = lax.broadcasted_iota(jnp.int32, (16, 128), 1)
    t = (i * 128 + j).astype(jnp.float32)
    ang = t * 2.0 * np.float32(np.pi) / 2048.0
    c = jnp.cos(ang)
    s = jnp.sin(ang)
    cos_ref[...] = c
    sin_ref[...] = s
    asc = jnp.abs(c) * _S
    ass = jnp.abs(s) * _S
    bx = jnp.where(asc > 0.0, 512.0 / jnp.maximum(asc, 1e-30), 3000.0)
    by = jnp.where(ass > 0.0, 2.0 / jnp.maximum(ass, 1e-30), 3000.0)
    bound = jnp.minimum(jnp.minimum(bx, by), 3000.0)
    nv_ref[...] = jnp.clip((bound * (1.0 / 16.0)).astype(jnp.int32) + 3,
                           1, _NCHUNK)


_trig = pl.pallas_call(
    _trig_kernel,
    out_shape=(jax.ShapeDtypeStruct((16, 128), jnp.float32),
               jax.ShapeDtypeStruct((16, 128), jnp.float32),
               jax.ShapeDtypeStruct((16, 128), jnp.int32)),
)

_mesh = plsc.VectorSubcoreMesh(core_axis_name="c", subcore_axis_name="s")


@functools.partial(
    pl.kernel,
    mesh=_mesh,
    out_type=jax.ShapeDtypeStruct((_CH, _H, _W), jnp.float32),
    scratch_types=[
        pltpu.VMEM((_CH, 4, 1024), jnp.float32),
        pltpu.VMEM((_RPW, 16), jnp.float32),
        pltpu.VMEM((_RPW, 16), jnp.float32),
        pltpu.VMEM((_RPW, 16), jnp.float32),
        pltpu.VMEM((2, _CH, 1, _W), jnp.float32),
        pltpu.SemaphoreType.DMA,
        pltpu.SemaphoreType.DMA,
    ],
    compiler_params=pltpu.CompilerParams(needs_layout_passes=False),
)
def _remap(data_hbm, tabs_hbm, out_hbm,
           table_v, cos_v, sin_v, nv_v, buf_v, sem0, sem1):
    wid = lax.axis_index("s") * 2 + lax.axis_index("c")
    base = wid * _RPW
    pltpu.sync_copy(data_hbm.at[:, pl.ds(0, 4), :], table_v)
    pltpu.sync_copy(tabs_hbm.at[pl.ds(base, _RPW)], cos_v)
    pltpu.sync_copy(tabs_hbm.at[pl.ds(_H + base, _RPW)], sin_v)
    pltpu.sync_copy(tabs_hbm.at[pl.ds(2 * _H + base, _RPW)], nv_v)
    iota16 = lax.iota(jnp.int32, 16)
    zeros16 = jnp.zeros((16,), jnp.float32)
    cidx = [jnp.full((16,), c, jnp.int32) for c in range(_CH)]
    sems = (sem0, sem1)

    def group(g, carry):
        carry = list(carry)
        for b in range(2):
            rl = 2 * g + b
            row = base + rl

            @pl.when(g > 0)
            def _wait():
                pltpu.make_async_copy(out_hbm.at[:, pl.ds(row, 1), :],
                                      buf_v.at[b], sems[b]).wait()

            def zchunk(k, c2):
                for c in range(_CH):
                    buf_v[b, c, 0, pl.ds(k * 16, 16)] = zeros16
                return c2

            lax.fori_loop(0, carry[b], zchunk, 0)

            cv = cos_v[rl]
            sv = sin_v[rl]
            nv = jnp.max(lax.bitcast_convert_type(nv_v[rl], jnp.int32))

            def chunk(k, c2):
                rrv = (iota16 + k * 16).astype(jnp.float32)
                rho = rrv * _S
                x = 512.0 + rho * cv
                y = 2.0 - rho * sv
                m = (x >= 0.0) & (x < 1024.0) & (y >= 0.0) & (y < 4.0)
                xi = jnp.clip(x.astype(jnp.int32), 0, 1023)
                yi = jnp.clip(y.astype(jnp.int32), 0, 3)
                for c in range(_CH):
                    val = plsc.load_gather(table_v, [cidx[c], yi, xi])
                    buf_v[b, c, 0, pl.ds(k * 16, 16)] = jnp.where(m, val, 0.0)
                return c2

            lax.fori_loop(0, nv, chunk, 0)
            pltpu.async_copy(buf_v.at[b], out_hbm.at[:, pl.ds(row, 1), :],
                             sems[b])
            carry[b] = nv
        return tuple(carry)

    lax.fori_loop(0, _RPW // 2, group,
                  (jnp.int32(_NCHUNK), jnp.int32(_NCHUNK)))
    for b in range(2):
        pltpu.make_async_copy(out_hbm.at[:, pl.ds(base, 1), :],
                              buf_v.at[b], sems[b]).wait()


def kernel(data):
    cos_t, sin_t, nv_t = _trig()
    nv_f = lax.bitcast_convert_type(nv_t, jnp.float32)
    tabs = jnp.concatenate(
        [cos_t.reshape(_H), sin_t.reshape(_H), nv_f.reshape(_H)])
    tabs_b = jnp.broadcast_to(tabs[:, None], (3 * _H, 16))
    return _remap(data, tabs_b)
